# baseline (speedup 1.0000x reference)
import functools

import jax
import jax.numpy as jnp
from jax import lax
from jax.experimental import pallas as pl
from jax.experimental.pallas import tpu as pltpu
from jax.experimental.pallas import tpu_sc as plsc

NODE_NUM = 512
DIM = 128
INPUT_DIM = 64
MOE = 8
RTK = 2
TOPK = 20
TAU = 1.0
B = 64
BN_ = B * NODE_NUM
NEG = -1e30


def _mm(a, b):
    return lax.dot_general(a, b, (((1,), (0,)), ((), ())),
                           preferred_element_type=jnp.float32)


def _mm_t(a, b):
    return lax.dot_general(a, b, (((1,), (1,)), ((), ())),
                           preferred_element_type=jnp.float32)


def _stage_a(data_ref, wc_ref, bc_ref, wap_ref, bap_ref, wav_ref, hsys_ref):
    x = data_ref[0]
    h = _mm(x, wc_ref[...]) + bc_ref[...]
    ah = _mm(h, wap_ref[...]) + bap_ref[...]
    ah = jnp.where(ah >= 0, ah, 0.2 * ah)
    e = _mm(ah, wav_ref[...])
    m = jnp.max(e, axis=0, keepdims=True)
    ex = jnp.exp(e - m)
    beta = ex / jnp.sum(ex, axis=0, keepdims=True)
    hsys_ref[0] = jnp.sum(beta * h, axis=0, keepdims=True)


def _stage_b(hsys_ref, wr_ref, br_ref, g_ref, psoft_ref, pt_ref):
    z = _mm(hsys_ref[...], wr_ref[...]) + br_ref[...]
    zg = (z + g_ref[...]) / TAU
    m = jnp.max(zg, axis=1, keepdims=True)
    ex = jnp.exp(zg - m)
    ps = ex / jnp.sum(ex, axis=1, keepdims=True)
    psoft_ref[...] = ps
    m1 = jnp.max(ps, axis=1, keepdims=True)
    nmax = jnp.sum(jnp.where(ps == m1, 1.0, 0.0), axis=1, keepdims=True)
    m2 = jnp.max(jnp.where(ps == m1, NEG, ps), axis=1, keepdims=True)
    thr = jnp.where(nmax > 1.5, m1, m2)
    ph = jnp.where(ps >= thr, ps, 0.0)
    pt_ref[...] = ph / jnp.maximum(jnp.sum(ph, axis=1, keepdims=True), 1e-12)


def _stage_c(u_ref, v_ref, eb_ref, pd_ref):
    pd_ref[0] = _mm(u_ref[0], v_ref[0]) + eb_ref[...]


def _stage_c2(pt_ref, pd_ref, mixed_ref):
    mixed_ref[...] = _mm(pt_ref[...], pd_ref[...])


SPLIT = 8
SC_CH = 4
SC_NW = 32
SC_ROWS = B * NODE_NUM // SPLIT
SC_RPW = SC_ROWS // SC_NW
SC_NCH = SC_RPW // SC_CH
SC_HALF = SC_NCH // 2
SC_HROWS = SC_HALF * SC_CH


def _sort16(v):
    s, _ = plsc.sort_key_val(v, v)
    return s


def _merge16(a, b):
    rb = lax.rev(b, (0,))
    return _sort16(jnp.minimum(a, rb)), _sort16(jnp.maximum(a, rb))


def _merge32_top(A, Bp, sort_hi=True):
    alo, ahi = A
    blo, bhi = Bp
    h1 = jnp.maximum(alo, lax.rev(bhi, (0,)))
    h2 = jnp.maximum(ahi, lax.rev(blo, (0,)))
    lo = _sort16(jnp.minimum(h1, h2))
    hi = jnp.maximum(h1, h2)
    return lo, (_sort16(hi) if sort_hi else hi)


def _sc_row(vrow_slices):
    segs = [_sort16(v) for v in vrow_slices]
    pairs = [_merge16(segs[2 * j], segs[2 * j + 1]) for j in range(16)]
    while len(pairs) > 2:
        pairs = [_merge32_top(pairs[j], pairs[j + 1])
                 for j in range(0, len(pairs), 2)]
    lo_s, hi = _merge32_top(pairs[0], pairs[1], sort_hi=False)
    rowmax = jnp.max(hi)
    lane = jnp.arange(16, dtype=jnp.int32)
    return jnp.where(lane == 0, jnp.full((16,), rowmax, jnp.float32), lo_s)


def _sc_topk_body(scores_hbm, tops_hbm, buf, obuf, sem0, sem1):
    wid = lax.axis_index("s") * 2 + lax.axis_index("c")
    base = wid * SC_RPW
    sems = (sem0, sem1)

    def start(i, slot):
        pltpu.async_copy(scores_hbm.at[pl.ds(base + i * SC_CH, SC_CH)],
                         buf.at[slot], sems[slot])

    def wait(slot):
        pltpu.make_async_copy(scores_hbm.at[pl.ds(base, SC_CH)],
                              buf.at[slot], sems[slot]).wait()

    def process(i_local, slot):
        for r in range(SC_CH):
            out16 = _sc_row([buf[slot, r, pl.ds(j * 16, 16)]
                             for j in range(32)])
            obuf[pl.ds(i_local * (SC_CH * 16) + r * 16, 16)] = out16

    def run_half(h):
        c0 = h * SC_HALF
        start(c0, 0)
        start(c0 + 1, 1)

        def step(j, carry):
            for slot in (0, 1):
                wait(slot)
                process(2 * j + slot, slot)

                @pl.when(2 * j + slot + 2 < SC_HALF)
                def _():
                    start(c0 + 2 * j + slot + 2, slot)
            return carry

        lax.fori_loop(0, SC_HALF // 2, step, 0)
        pltpu.sync_copy(obuf,
                        tops_hbm.at[pl.ds((base + h * SC_HROWS) * 16,
                                          SC_HROWS * 16)])

    run_half(0)
    run_half(1)


def _sc_topk(scores2d):
    mesh = plsc.VectorSubcoreMesh(core_axis_name="c", subcore_axis_name="s")
    return pl.kernel(
        _sc_topk_body,
        out_type=jax.ShapeDtypeStruct((SC_ROWS * 16,), jnp.float32),
        mesh=mesh,
        compiler_params=pltpu.CompilerParams(needs_layout_passes=False),
        scratch_types=[
            pltpu.VMEM((2, SC_CH, NODE_NUM), jnp.float32),
            pltpu.VMEM((SC_HROWS * 16,), jnp.float32),
            pltpu.SemaphoreType.DMA,
            pltpu.SemaphoreType.DMA,
        ],
    )(scores2d)


def _stage_d1(mixed_ref, scores_ref):
    mx = mixed_ref[0]
    scores_ref[...] = _mm_t(mx, mx)


def _stage_d(data_ref, mixed_ref, tops_ref, wl_ref, ai_ref, aj_ref, aei_ref,
             aej_ref, gb_ref, agg_ref, s1_ref, ss1_ref):
    b = pl.program_id(0)
    x = data_ref[0]
    mx = mixed_ref[0]
    xl = _mm(x, wl_ref[...])
    a_i = _mm(xl, ai_ref[...]) + _mm(mx, aei_ref[...])
    a_j = _mm_t(aj_ref[...], xl) + _mm_t(aej_ref[...], mx)
    scores = _mm_t(mx, mx)
    tops = tops_ref[0]
    t = tops[:, 12:13]
    rowmax = tops[:, 0:1]
    mask = scores >= t
    ews = jnp.where(mask, jnp.exp(scores - rowmax), 0.0)
    sw = jnp.sum(ews, axis=1, keepdims=True)
    alpha = a_i + a_j
    alpha = jnp.where(alpha >= 0, alpha, 0.2 * alpha)
    amax = jnp.max(jnp.where(mask, alpha, NEG), axis=1, keepdims=True)
    exa = jnp.where(mask, jnp.exp(alpha - amax), 0.0)
    den = jnp.sum(exa, axis=1, keepdims=True)
    scale = 1.0 / ((den + 1e-16) * sw)
    wmat = exa * ews * scale
    agg = _mm(wmat, xl) + gb_ref[...]
    agg_ref[0] = agg

    @pl.when(b == 0)
    def _():
        s1_ref[...] = jnp.zeros_like(s1_ref)
        ss1_ref[...] = jnp.zeros_like(ss1_ref)

    s1_ref[...] += jnp.sum(agg, axis=0, keepdims=True)
    ss1_ref[...] += jnp.sum(agg * agg, axis=0, keepdims=True)


def _bn1_pre(a, s1cat, ss1cat, g1, b1, emb):
    mean = jnp.sum(s1cat, axis=0, keepdims=True) * (1.0 / BN_)
    var = jnp.sum(ss1cat, axis=0, keepdims=True) * (1.0 / BN_) - mean * mean
    inv = lax.rsqrt(var + 1e-5)
    y = jnp.maximum((a - mean) * inv * g1 + b1, 0.0)
    return y * emb


def _stage_e(agg_ref, s1_ref, ss1_ref, g1_ref, b1_ref,
             emb_ref, s2_ref, ss2_ref):
    b = pl.program_id(0)
    pre = _bn1_pre(agg_ref[0], s1_ref[...], ss1_ref[...],
                   g1_ref[...], b1_ref[...], emb_ref[...])

    @pl.when(b == 0)
    def _():
        s2_ref[...] = jnp.zeros_like(s2_ref)
        ss2_ref[...] = jnp.zeros_like(ss2_ref)

    s2_ref[...] += jnp.sum(pre, axis=0, keepdims=True)
    ss2_ref[...] += jnp.sum(pre * pre, axis=0, keepdims=True)


def _stage_f(agg_ref, s1_ref, ss1_ref, s2_ref, ss2_ref,
             g1_ref, b1_ref, emb_ref, g2_ref, b2_ref,
             wo_ref, bo_ref, out_ref):
    p = _bn1_pre(agg_ref[0], s1_ref[...], ss1_ref[...],
                 g1_ref[...], b1_ref[...], emb_ref[...])
    mean = jnp.sum(s2_ref[...], axis=0, keepdims=True) * (1.0 / BN_)
    var = (jnp.sum(ss2_ref[...], axis=0, keepdims=True) * (1.0 / BN_)
           - mean * mean)
    inv = lax.rsqrt(var + 1e-5)
    y = (p - mean) * inv * g2_ref[...] + b2_ref[...]
    y = jnp.maximum(y, 0.0)
    out_ref[0] = _mm_t(wo_ref[...], y) + bo_ref[...]


def kernel(data, org_edge_index, emb_table, e_base, low_rank_u, low_rank_v,
           W_cond, b_cond, W_ap, b_ap, w_av, W_r, b_r, W_lin, att_i, att_j,
           att_em_i, att_em_j, gnn_bias, bn1_g, bn1_b, bn2_g, bn2_b, W_out,
           b_out):
    f32 = jnp.float32
    N, D, F, M = NODE_NUM, DIM, INPUT_DIM, MOE
    row = lambda v: v.reshape(1, -1).astype(f32)
    col = lambda v: v.reshape(-1, 1).astype(f32)

    u = jnp.clip(jax.random.uniform(jax.random.key(42), (B, M), f32),
                 1e-6, 1.0 - 1e-6)
    g_const = -jnp.log(-jnp.log(u))

    h_sys = pl.pallas_call(
        _stage_a,
        grid=(B,),
        in_specs=[
            pl.BlockSpec((1, N, F), lambda b: (b, 0, 0)),
            pl.BlockSpec((F, D), lambda b: (0, 0)),
            pl.BlockSpec((1, D), lambda b: (0, 0)),
            pl.BlockSpec((D, D), lambda b: (0, 0)),
            pl.BlockSpec((1, D), lambda b: (0, 0)),
            pl.BlockSpec((D, 1), lambda b: (0, 0)),
        ],
        out_specs=pl.BlockSpec((1, 1, D), lambda b: (b, 0, 0)),
        out_shape=jax.ShapeDtypeStruct((B, 1, D), f32),
    )(data, W_cond, row(b_cond), W_ap, row(b_ap), col(w_av))
    h_sys = h_sys.reshape(B, D)

    pi_soft, pi_t = pl.pallas_call(
        _stage_b,
        in_specs=[pl.BlockSpec((B, D), lambda: (0, 0)),
                  pl.BlockSpec((D, M), lambda: (0, 0)),
                  pl.BlockSpec((1, M), lambda: (0, 0)),
                  pl.BlockSpec((B, M), lambda: (0, 0))],
        out_specs=[pl.BlockSpec((B, M), lambda: (0, 0)),
                   pl.BlockSpec((B, M), lambda: (0, 0))],
        out_shape=[jax.ShapeDtypeStruct((B, M), f32),
                   jax.ShapeDtypeStruct((B, M), f32)],
    )(h_sys, W_r, row(b_r), g_const)

    pd = pl.pallas_call(
        _stage_c,
        grid=(M,),
        in_specs=[pl.BlockSpec((1, N, 8), lambda m: (m, 0, 0)),
                  pl.BlockSpec((1, 8, D), lambda m: (m, 0, 0)),
                  pl.BlockSpec((N, D), lambda m: (0, 0))],
        out_specs=pl.BlockSpec((1, N, D), lambda m: (m, 0, 0)),
        out_shape=jax.ShapeDtypeStruct((M, N, D), f32),
    )(low_rank_u, low_rank_v, e_base)

    CH = 4096
    NC = N * D // CH
    mixed_flat = pl.pallas_call(
        _stage_c2,
        grid=(NC,),
        in_specs=[pl.BlockSpec((B, M), lambda c: (0, 0)),
                  pl.BlockSpec((M, CH), lambda c: (0, c))],
        out_specs=pl.BlockSpec((B, CH), lambda c: (0, c)),
        out_shape=jax.ShapeDtypeStruct((B, N * D), f32),
    )(pi_t, pd.reshape(M, N * D))
    mixed = mixed_flat.reshape(B, N, D)

    H = B // SPLIT
    halves = []
    for h in range(SPLIT):
        scores2d = pl.pallas_call(
            _stage_d1,
            grid=(H,),
            in_specs=[pl.BlockSpec((1, N, D),
                                   lambda b, h=h: (b + h * H, 0, 0))],
            out_specs=pl.BlockSpec((N, N), lambda b: (b, 0)),
            out_shape=jax.ShapeDtypeStruct((H * N, N), f32),
        )(mixed)
        tops = _sc_topk(scores2d).reshape(H, N, 16)
        agg_h, s1_h, ss1_h = pl.pallas_call(
            _stage_d,
            grid=(H,),
            in_specs=[
                pl.BlockSpec((1, N, F), lambda b, h=h: (b + h * H, 0, 0)),
                pl.BlockSpec((1, N, D), lambda b, h=h: (b + h * H, 0, 0)),
                pl.BlockSpec((1, N, 16), lambda b: (b, 0, 0)),
                pl.BlockSpec((F, D), lambda b: (0, 0)),
                pl.BlockSpec((D, 1), lambda b: (0, 0)),
                pl.BlockSpec((1, D), lambda b: (0, 0)),
                pl.BlockSpec((D, 1), lambda b: (0, 0)),
                pl.BlockSpec((1, D), lambda b: (0, 0)),
                pl.BlockSpec((1, D), lambda b: (0, 0)),
            ],
            out_specs=[pl.BlockSpec((1, N, D), lambda b: (b, 0, 0)),
                       pl.BlockSpec((1, D), lambda b: (0, 0)),
                       pl.BlockSpec((1, D), lambda b: (0, 0))],
            out_shape=[jax.ShapeDtypeStruct((H, N, D), f32),
                       jax.ShapeDtypeStruct((1, D), f32),
                       jax.ShapeDtypeStruct((1, D), f32)],
        )(data, mixed, tops, W_lin, col(att_i), row(att_j), col(att_em_i),
          row(att_em_j), row(gnn_bias))
        halves.append((agg_h, s1_h, ss1_h))
    s1cat = jnp.concatenate([hh[1] for hh in halves], axis=0)
    ss1cat = jnp.concatenate([hh[2] for hh in halves], axis=0)

    vec_spec = pl.BlockSpec((1, D), lambda b: (0, 0))
    cat_spec = pl.BlockSpec((SPLIT, D), lambda b: (0, 0))
    emb_spec = pl.BlockSpec((N, D), lambda b: (0, 0))

    stats2 = []
    for agg_h, _, _ in halves:
        s2_h, ss2_h = pl.pallas_call(
            _stage_e,
            grid=(H,),
            in_specs=[pl.BlockSpec((1, N, D), lambda b: (b, 0, 0)),
                      cat_spec, cat_spec, vec_spec, vec_spec, emb_spec],
            out_specs=[vec_spec, vec_spec],
            out_shape=[jax.ShapeDtypeStruct((1, D), f32),
                       jax.ShapeDtypeStruct((1, D), f32)],
        )(agg_h, s1cat, ss1cat, row(bn1_g), row(bn1_b), emb_table)
        stats2.append((s2_h, ss2_h))
    s2cat = jnp.concatenate([ss[0] for ss in stats2], axis=0)
    ss2cat = jnp.concatenate([ss[1] for ss in stats2], axis=0)

    outs = []
    for agg_h, _, _ in halves:
        out_h = pl.pallas_call(
            _stage_f,
            grid=(H,),
            in_specs=[pl.BlockSpec((1, N, D), lambda b: (b, 0, 0)),
                      cat_spec, cat_spec, cat_spec, cat_spec,
                      vec_spec, vec_spec, emb_spec, vec_spec, vec_spec,
                      vec_spec, pl.BlockSpec((1, 1), lambda b: (0, 0))],
            out_specs=pl.BlockSpec((1, 1, N), lambda b: (b, 0, 0)),
            out_shape=jax.ShapeDtypeStruct((H, 1, N), f32),
        )(agg_h, s1cat, ss1cat, s2cat, ss2cat,
          row(bn1_g), row(bn1_b), emb_table, row(bn2_g), row(bn2_b),
          row(W_out), b_out.reshape(1, 1))
        outs.append(out_h.reshape(H, N))

    return jnp.concatenate(outs, axis=0), h_sys, pi_soft


# speedup vs baseline: 1.0801x; 1.0801x over previous
import functools

import jax
import jax.numpy as jnp
from jax import lax
from jax.experimental import pallas as pl
from jax.experimental.pallas import tpu as pltpu
from jax.experimental.pallas import tpu_sc as plsc

NODE_NUM = 512
DIM = 128
INPUT_DIM = 64
MOE = 8
RTK = 2
TOPK = 20
TAU = 1.0
B = 64
BN_ = B * NODE_NUM
NEG = -1e30


def _mm(a, b):
    return lax.dot_general(a, b, (((1,), (0,)), ((), ())),
                           preferred_element_type=jnp.float32)


def _mm_t(a, b):
    return lax.dot_general(a, b, (((1,), (1,)), ((), ())),
                           preferred_element_type=jnp.float32)


def _stage_a(data_ref, wc_ref, bc_ref, wap_ref, bap_ref, wav_ref, hsys_ref):
    x = data_ref[0]
    h = _mm(x, wc_ref[...]) + bc_ref[...]
    ah = _mm(h, wap_ref[...]) + bap_ref[...]
    ah = jnp.where(ah >= 0, ah, 0.2 * ah)
    e = _mm(ah, wav_ref[...])
    m = jnp.max(e, axis=0, keepdims=True)
    ex = jnp.exp(e - m)
    beta = ex / jnp.sum(ex, axis=0, keepdims=True)
    hsys_ref[0] = jnp.sum(beta * h, axis=0, keepdims=True)


def _stage_b(hsys_ref, wr_ref, br_ref, g_ref, psoft_ref, pt_ref):
    z = _mm(hsys_ref[...], wr_ref[...]) + br_ref[...]
    zg = (z + g_ref[...]) / TAU
    m = jnp.max(zg, axis=1, keepdims=True)
    ex = jnp.exp(zg - m)
    ps = ex / jnp.sum(ex, axis=1, keepdims=True)
    psoft_ref[...] = ps
    m1 = jnp.max(ps, axis=1, keepdims=True)
    nmax = jnp.sum(jnp.where(ps == m1, 1.0, 0.0), axis=1, keepdims=True)
    m2 = jnp.max(jnp.where(ps == m1, NEG, ps), axis=1, keepdims=True)
    thr = jnp.where(nmax > 1.5, m1, m2)
    ph = jnp.where(ps >= thr, ps, 0.0)
    pt_ref[...] = ph / jnp.maximum(jnp.sum(ph, axis=1, keepdims=True), 1e-12)


def _stage_c(u_ref, v_ref, eb_ref, pd_ref):
    pd_ref[0] = _mm(u_ref[0], v_ref[0]) + eb_ref[...]


def _stage_c2(pt_ref, pd_ref, mixed_ref):
    mixed_ref[...] = _mm(pt_ref[...], pd_ref[...])


SPLIT = 4
SC_CH = 4
SC_NW = 32
SC_ROWS = B * NODE_NUM // SPLIT
SC_RPW = SC_ROWS // SC_NW
SC_NCH = SC_RPW // SC_CH
SC_HALF = SC_NCH // 2
SC_HROWS = SC_HALF * SC_CH


def _sort16(v):
    s, _ = plsc.sort_key_val(v, v)
    return s


def _merge16(a, b):
    rb = lax.rev(b, (0,))
    return _sort16(jnp.minimum(a, rb)), _sort16(jnp.maximum(a, rb))


def _merge32_top(A, Bp, sort_hi=True):
    alo, ahi = A
    blo, bhi = Bp
    h1 = jnp.maximum(alo, lax.rev(bhi, (0,)))
    h2 = jnp.maximum(ahi, lax.rev(blo, (0,)))
    lo = _sort16(jnp.minimum(h1, h2))
    hi = jnp.maximum(h1, h2)
    return lo, (_sort16(hi) if sort_hi else hi)


def _sc_row(vrow_slices):
    segs = [_sort16(v) for v in vrow_slices]
    pairs = [_merge16(segs[2 * j], segs[2 * j + 1]) for j in range(16)]
    while len(pairs) > 2:
        pairs = [_merge32_top(pairs[j], pairs[j + 1])
                 for j in range(0, len(pairs), 2)]
    lo_s, hi = _merge32_top(pairs[0], pairs[1], sort_hi=False)
    rowmax = jnp.max(hi)
    lane = jnp.arange(16, dtype=jnp.int32)
    return jnp.where(lane == 0, jnp.full((16,), rowmax, jnp.float32), lo_s)


def _sc_topk_body(scores_hbm, tops_hbm, buf, obuf, sem0, sem1):
    wid = lax.axis_index("s") * 2 + lax.axis_index("c")
    base = wid * SC_RPW
    sems = (sem0, sem1)

    def start(i, slot):
        pltpu.async_copy(scores_hbm.at[pl.ds(base + i * SC_CH, SC_CH)],
                         buf.at[slot], sems[slot])

    def wait(slot):
        pltpu.make_async_copy(scores_hbm.at[pl.ds(base, SC_CH)],
                              buf.at[slot], sems[slot]).wait()

    def process(i_local, slot):
        for r in range(SC_CH):
            out16 = _sc_row([buf[slot, r, pl.ds(j * 16, 16)]
                             for j in range(32)])
            obuf[pl.ds(i_local * (SC_CH * 16) + r * 16, 16)] = out16

    def run_half(h):
        c0 = h * SC_HALF
        start(c0, 0)
        start(c0 + 1, 1)

        def step(j, carry):
            for slot in (0, 1):
                wait(slot)
                process(2 * j + slot, slot)

                @pl.when(2 * j + slot + 2 < SC_HALF)
                def _():
                    start(c0 + 2 * j + slot + 2, slot)
            return carry

        lax.fori_loop(0, SC_HALF // 2, step, 0)
        pltpu.sync_copy(obuf,
                        tops_hbm.at[pl.ds((base + h * SC_HROWS) * 16,
                                          SC_HROWS * 16)])

    run_half(0)
    run_half(1)


def _sc_topk(scores2d):
    mesh = plsc.VectorSubcoreMesh(core_axis_name="c", subcore_axis_name="s")
    return pl.kernel(
        _sc_topk_body,
        out_type=jax.ShapeDtypeStruct((SC_ROWS * 16,), jnp.float32),
        mesh=mesh,
        compiler_params=pltpu.CompilerParams(needs_layout_passes=False),
        scratch_types=[
            pltpu.VMEM((2, SC_CH, NODE_NUM), jnp.float32),
            pltpu.VMEM((SC_HROWS * 16,), jnp.float32),
            pltpu.SemaphoreType.DMA,
            pltpu.SemaphoreType.DMA,
        ],
    )(scores2d)


def _stage_d1(mixed_ref, scores_ref):
    mx = mixed_ref[0]
    scores_ref[...] = _mm_t(mx, mx)


def _stage_d(data_ref, mixed_ref, tops_ref, wl_ref, ai_ref, aj_ref, aei_ref,
             aej_ref, gb_ref, agg_ref, s1_ref, ss1_ref):
    b = pl.program_id(0)
    x = data_ref[0]
    mx = mixed_ref[0]
    xl = _mm(x, wl_ref[...])
    a_i = _mm(xl, ai_ref[...]) + _mm(mx, aei_ref[...])
    a_j = _mm_t(aj_ref[...], xl) + _mm_t(aej_ref[...], mx)
    scores = _mm_t(mx, mx)
    tops = tops_ref[0]
    t = tops[:, 12:13]
    rowmax = tops[:, 0:1]
    mask = scores >= t
    ews = jnp.where(mask, jnp.exp(scores - rowmax), 0.0)
    sw = jnp.sum(ews, axis=1, keepdims=True)
    alpha = a_i + a_j
    alpha = jnp.where(alpha >= 0, alpha, 0.2 * alpha)
    amax = jnp.max(jnp.where(mask, alpha, NEG), axis=1, keepdims=True)
    exa = jnp.where(mask, jnp.exp(alpha - amax), 0.0)
    den = jnp.sum(exa, axis=1, keepdims=True)
    scale = 1.0 / ((den + 1e-16) * sw)
    wmat = exa * ews * scale
    agg = _mm(wmat, xl) + gb_ref[...]
    agg_ref[0] = agg

    @pl.when(b == 0)
    def _():
        s1_ref[...] = jnp.zeros_like(s1_ref)
        ss1_ref[...] = jnp.zeros_like(ss1_ref)

    s1_ref[...] += jnp.sum(agg, axis=0, keepdims=True)
    ss1_ref[...] += jnp.sum(agg * agg, axis=0, keepdims=True)


def _bn1_pre(a, s1cat, ss1cat, g1, b1, emb):
    mean = jnp.sum(s1cat, axis=0, keepdims=True) * (1.0 / BN_)
    var = jnp.sum(ss1cat, axis=0, keepdims=True) * (1.0 / BN_) - mean * mean
    inv = lax.rsqrt(var + 1e-5)
    y = jnp.maximum((a - mean) * inv * g1 + b1, 0.0)
    return y * emb


def _stage_e(agg_ref, s1_ref, ss1_ref, g1_ref, b1_ref,
             emb_ref, s2_ref, ss2_ref):
    b = pl.program_id(0)
    pre = _bn1_pre(agg_ref[0], s1_ref[...], ss1_ref[...],
                   g1_ref[...], b1_ref[...], emb_ref[...])

    @pl.when(b == 0)
    def _():
        s2_ref[...] = jnp.zeros_like(s2_ref)
        ss2_ref[...] = jnp.zeros_like(ss2_ref)

    s2_ref[...] += jnp.sum(pre, axis=0, keepdims=True)
    ss2_ref[...] += jnp.sum(pre * pre, axis=0, keepdims=True)


def _stage_f(agg_ref, s1_ref, ss1_ref, s2_ref, ss2_ref,
             g1_ref, b1_ref, emb_ref, g2_ref, b2_ref,
             wo_ref, bo_ref, out_ref):
    p = _bn1_pre(agg_ref[0], s1_ref[...], ss1_ref[...],
                 g1_ref[...], b1_ref[...], emb_ref[...])
    mean = jnp.sum(s2_ref[...], axis=0, keepdims=True) * (1.0 / BN_)
    var = (jnp.sum(ss2_ref[...], axis=0, keepdims=True) * (1.0 / BN_)
           - mean * mean)
    inv = lax.rsqrt(var + 1e-5)
    y = (p - mean) * inv * g2_ref[...] + b2_ref[...]
    y = jnp.maximum(y, 0.0)
    out_ref[0] = _mm_t(wo_ref[...], y) + bo_ref[...]


def kernel(data, org_edge_index, emb_table, e_base, low_rank_u, low_rank_v,
           W_cond, b_cond, W_ap, b_ap, w_av, W_r, b_r, W_lin, att_i, att_j,
           att_em_i, att_em_j, gnn_bias, bn1_g, bn1_b, bn2_g, bn2_b, W_out,
           b_out):
    f32 = jnp.float32
    N, D, F, M = NODE_NUM, DIM, INPUT_DIM, MOE
    row = lambda v: v.reshape(1, -1).astype(f32)
    col = lambda v: v.reshape(-1, 1).astype(f32)

    u = jnp.clip(jax.random.uniform(jax.random.key(42), (B, M), f32),
                 1e-6, 1.0 - 1e-6)
    g_const = -jnp.log(-jnp.log(u))

    h_sys = pl.pallas_call(
        _stage_a,
        grid=(B,),
        in_specs=[
            pl.BlockSpec((1, N, F), lambda b: (b, 0, 0)),
            pl.BlockSpec((F, D), lambda b: (0, 0)),
            pl.BlockSpec((1, D), lambda b: (0, 0)),
            pl.BlockSpec((D, D), lambda b: (0, 0)),
            pl.BlockSpec((1, D), lambda b: (0, 0)),
            pl.BlockSpec((D, 1), lambda b: (0, 0)),
        ],
        out_specs=pl.BlockSpec((1, 1, D), lambda b: (b, 0, 0)),
        out_shape=jax.ShapeDtypeStruct((B, 1, D), f32),
    )(data, W_cond, row(b_cond), W_ap, row(b_ap), col(w_av))
    h_sys = h_sys.reshape(B, D)

    pi_soft, pi_t = pl.pallas_call(
        _stage_b,
        in_specs=[pl.BlockSpec((B, D), lambda: (0, 0)),
                  pl.BlockSpec((D, M), lambda: (0, 0)),
                  pl.BlockSpec((1, M), lambda: (0, 0)),
                  pl.BlockSpec((B, M), lambda: (0, 0))],
        out_specs=[pl.BlockSpec((B, M), lambda: (0, 0)),
                   pl.BlockSpec((B, M), lambda: (0, 0))],
        out_shape=[jax.ShapeDtypeStruct((B, M), f32),
                   jax.ShapeDtypeStruct((B, M), f32)],
    )(h_sys, W_r, row(b_r), g_const)

    pd = pl.pallas_call(
        _stage_c,
        grid=(M,),
        in_specs=[pl.BlockSpec((1, N, 8), lambda m: (m, 0, 0)),
                  pl.BlockSpec((1, 8, D), lambda m: (m, 0, 0)),
                  pl.BlockSpec((N, D), lambda m: (0, 0))],
        out_specs=pl.BlockSpec((1, N, D), lambda m: (m, 0, 0)),
        out_shape=jax.ShapeDtypeStruct((M, N, D), f32),
    )(low_rank_u, low_rank_v, e_base)

    CH = 4096
    NC = N * D // CH
    mixed_flat = pl.pallas_call(
        _stage_c2,
        grid=(NC,),
        in_specs=[pl.BlockSpec((B, M), lambda c: (0, 0)),
                  pl.BlockSpec((M, CH), lambda c: (0, c))],
        out_specs=pl.BlockSpec((B, CH), lambda c: (0, c)),
        out_shape=jax.ShapeDtypeStruct((B, N * D), f32),
    )(pi_t, pd.reshape(M, N * D))
    mixed = mixed_flat.reshape(B, N, D)

    H = B // SPLIT
    halves = []
    for h in range(SPLIT):
        scores2d = pl.pallas_call(
            _stage_d1,
            grid=(H,),
            in_specs=[pl.BlockSpec((1, N, D),
                                   lambda b, h=h: (b + h * H, 0, 0))],
            out_specs=pl.BlockSpec((N, N), lambda b: (b, 0)),
            out_shape=jax.ShapeDtypeStruct((H * N, N), f32),
        )(mixed)
        tops = _sc_topk(scores2d).reshape(H, N, 16)
        agg_h, s1_h, ss1_h = pl.pallas_call(
            _stage_d,
            grid=(H,),
            in_specs=[
                pl.BlockSpec((1, N, F), lambda b, h=h: (b + h * H, 0, 0)),
                pl.BlockSpec((1, N, D), lambda b, h=h: (b + h * H, 0, 0)),
                pl.BlockSpec((1, N, 16), lambda b: (b, 0, 0)),
                pl.BlockSpec((F, D), lambda b: (0, 0)),
                pl.BlockSpec((D, 1), lambda b: (0, 0)),
                pl.BlockSpec((1, D), lambda b: (0, 0)),
                pl.BlockSpec((D, 1), lambda b: (0, 0)),
                pl.BlockSpec((1, D), lambda b: (0, 0)),
                pl.BlockSpec((1, D), lambda b: (0, 0)),
            ],
            out_specs=[pl.BlockSpec((1, N, D), lambda b: (b, 0, 0)),
                       pl.BlockSpec((1, D), lambda b: (0, 0)),
                       pl.BlockSpec((1, D), lambda b: (0, 0))],
            out_shape=[jax.ShapeDtypeStruct((H, N, D), f32),
                       jax.ShapeDtypeStruct((1, D), f32),
                       jax.ShapeDtypeStruct((1, D), f32)],
        )(data, mixed, tops, W_lin, col(att_i), row(att_j), col(att_em_i),
          row(att_em_j), row(gnn_bias))
        halves.append((agg_h, s1_h, ss1_h))
    s1cat = jnp.concatenate([hh[1] for hh in halves], axis=0)
    ss1cat = jnp.concatenate([hh[2] for hh in halves], axis=0)

    vec_spec = pl.BlockSpec((1, D), lambda b: (0, 0))
    cat_spec = pl.BlockSpec((SPLIT, D), lambda b: (0, 0))
    emb_spec = pl.BlockSpec((N, D), lambda b: (0, 0))

    stats2 = []
    for agg_h, _, _ in halves:
        s2_h, ss2_h = pl.pallas_call(
            _stage_e,
            grid=(H,),
            in_specs=[pl.BlockSpec((1, N, D), lambda b: (b, 0, 0)),
                      cat_spec, cat_spec, vec_spec, vec_spec, emb_spec],
            out_specs=[vec_spec, vec_spec],
            out_shape=[jax.ShapeDtypeStruct((1, D), f32),
                       jax.ShapeDtypeStruct((1, D), f32)],
        )(agg_h, s1cat, ss1cat, row(bn1_g), row(bn1_b), emb_table)
        stats2.append((s2_h, ss2_h))
    s2cat = jnp.concatenate([ss[0] for ss in stats2], axis=0)
    ss2cat = jnp.concatenate([ss[1] for ss in stats2], axis=0)

    outs = []
    for agg_h, _, _ in halves:
        out_h = pl.pallas_call(
            _stage_f,
            grid=(H,),
            in_specs=[pl.BlockSpec((1, N, D), lambda b: (b, 0, 0)),
                      cat_spec, cat_spec, cat_spec, cat_spec,
                      vec_spec, vec_spec, emb_spec, vec_spec, vec_spec,
                      vec_spec, pl.BlockSpec((1, 1), lambda b: (0, 0))],
            out_specs=pl.BlockSpec((1, 1, N), lambda b: (b, 0, 0)),
            out_shape=jax.ShapeDtypeStruct((H, 1, N), f32),
        )(agg_h, s1cat, ss1cat, s2cat, ss2cat,
          row(bn1_g), row(bn1_b), emb_table, row(bn2_g), row(bn2_b),
          row(W_out), b_out.reshape(1, 1))
        outs.append(out_h.reshape(H, N))

    return jnp.concatenate(outs, axis=0), h_sys, pi_soft
